# baseline (speedup 1.0000x reference)
import jax
import jax.numpy as jnp
from jax.experimental import pallas as pl

_BR = 512


def _body(s_ref, m_ref, o_ref):
    s = s_ref[...]
    m = m_ref[...]
    o_ref[...] = jnp.where(m[None, :, :], jnp.float32(-jnp.inf), s[:, None, :])


def kernel(scores, M):
    B, C = scores.shape
    L = M.shape[0]
    return pl.pallas_call(
        _body,
        grid=(B // _BR,),
        in_specs=[
            pl.BlockSpec((_BR, C), lambda i: (i, 0)),
            pl.BlockSpec((L, C), lambda i: (0, 0)),
        ],
        out_specs=pl.BlockSpec((_BR, L, C), lambda i: (i, 0, 0)),
        out_shape=jax.ShapeDtypeStruct((B, L, C), scores.dtype),
    )(scores, M)
